# Initial kernel scaffold; baseline (speedup 1.0000x reference)
import jax
import jax.numpy as jnp
from jax.experimental import pallas as pl


def kernel(x, table, W, b):
    raise NotImplementedError("write your pallas kernel here")


# speedup vs baseline: 8.2478x; 8.2478x over previous
import functools

import jax
import jax.numpy as jnp
from jax import lax
from jax.experimental import pallas as pl
from jax.experimental.pallas import tpu as pltpu
from jax.experimental.pallas import tpu_sc as plsc

B = 16384
S = 200
D = 16
C = 8

NC = 2
NS = 16
NW = NC * NS
EPW = B // NW
CH = 16
NROW = CH * S // 128
NCHUNK = EPW // CH
XROWS_PER_W = EPW * S // 128


def _sc_pool_body(xi_hbm, table_hbm, pooled_hbm, idx_v, rows_v, pool_v, sem):
    wid = lax.axis_index("s") * NC + lax.axis_index("c")
    base_elem = wid * EPW
    base_idx = wid * (EPW * S)
    scale = jnp.float32(1.0 / S)

    def chunk_body(c, _):
        pltpu.sync_copy(xi_hbm.at[pl.ds(base_idx + c * (CH * S), CH * S)], idx_v)
        cps = [
            pltpu.async_copy(
                table_hbm.at[idx_v.at[pl.ds(j * 128, 128)]],
                rows_v.at[pl.ds(j * 128, 128)],
                sem,
            )
            for j in range(NROW)
        ]
        for cp in cps:
            cp.wait()
        for e in range(CH):
            r0 = e * S

            def acc8(i, accs):
                a0, a1, a2, a3 = accs
                r = r0 + i * 8
                a0 = a0 + rows_v[r]
                a1 = a1 + rows_v[r + 1]
                a2 = a2 + rows_v[r + 2]
                a3 = a3 + rows_v[r + 3]
                a0 = a0 + rows_v[r + 4]
                a1 = a1 + rows_v[r + 5]
                a2 = a2 + rows_v[r + 6]
                a3 = a3 + rows_v[r + 7]
                return (a0, a1, a2, a3)

            z = jnp.zeros((16,), jnp.float32)
            a0, a1, a2, a3 = lax.fori_loop(0, S // 8, acc8, (z, z, z, z))
            pool_v[c * CH + e] = ((a0 + a1) + (a2 + a3)) * scale
        return 0

    lax.fori_loop(0, NCHUNK, chunk_body, 0)
    pltpu.sync_copy(pool_v, pooled_hbm.at[pl.ds(base_elem, EPW)])


@functools.partial(jax.jit, static_argnames=())
def _sc_pool(xi, table):
    mesh = plsc.VectorSubcoreMesh(core_axis_name="c", subcore_axis_name="s")
    return pl.kernel(
        _sc_pool_body,
        out_type=jax.ShapeDtypeStruct((B, D), jnp.float32),
        mesh=mesh,
        scratch_types=[
            pltpu.VMEM((CH * S,), jnp.int32),
            pltpu.VMEM((CH * S, D), jnp.float32),
            pltpu.VMEM((EPW, D), jnp.float32),
            pltpu.SemaphoreType.DMA,
        ],
        compiler_params=pltpu.CompilerParams(use_tc_tiling_on_sc=False),
    )(xi, table)


def _tc_linear_body(p_ref, w_ref, b_ref, o_ref):
    o_ref[...] = (
        jnp.dot(p_ref[...], w_ref[...], preferred_element_type=jnp.float32)
        + b_ref[...]
    )


def kernel(x, table, W, b):
    xi = x.reshape(B * S)
    pooled = _sc_pool(xi, table)
    return pl.pallas_call(
        _tc_linear_body,
        out_shape=jax.ShapeDtypeStruct((B, C), jnp.float32),
    )(pooled, W, b.reshape(1, C))
